# baseline (speedup 1.0000x reference)
import functools

import jax
import jax.numpy as jnp
from jax import lax
from jax.experimental import pallas as pl
from jax.experimental.pallas import tpu as pltpu
from jax.experimental.pallas import tpu_sc as plsc

B, L, C, T = 16, 512, 32, 4
D, R, NR, ND, K = 512, 128, 3, 3, 25
MB = 4
G = B // MB


def _stage1_body(x_ref, m_ref, xk_ref, Wv_ref, Wm_ref, be_ref,
                 reg_ref, Wp_ref, bp_ref,
                 Wft_ref, Wfb_ref, Ws_ref, Wt_ref, bf_ref, bs_ref, bt_ref,
                 Wc1_ref, bc1_ref, Wc2_ref, bc2_ref,
                 s_ref, df_ref, d2_ref, dp_ref, reg3_ref,
                 A_s, W2_s):
    g = pl.program_id(0)

    @pl.when(g == 0)
    def _fold():
        Ws = Ws_ref[...]
        Wt = Wt_ref[...]
        A_s[...] = Wft_ref[...] @ jnp.concatenate([Ws, Wt - Ws], axis=1)
        Qm = Wfb_ref[...] @ Wt
        reg3 = reg_ref[...] @ Qm + (bf_ref[...] @ Wt + bs_ref[...] + bt_ref[...])
        reg3_ref[...] = jnp.concatenate(
            [reg3, jnp.zeros((R, 128 - C), jnp.float32)], axis=1)
        li = lax.broadcasted_iota(jnp.int32, (L, L), 0)
        mi = lax.broadcasted_iota(jnp.int32, (L, L), 1)
        band = ((mi >= li - 12) & (mi <= li + 12)).astype(jnp.float32)
        ex0 = jnp.where(mi == 0, jnp.maximum(12 - li, 0), 0).astype(jnp.float32)
        ex1 = jnp.where(mi == L - 1, jnp.maximum(li - (L - 13), 0), 0).astype(jnp.float32)
        W2_s[...] = (band + ex0 + ex1) * (1.0 / K)

    xm = (x_ref[...] * m_ref[...]).reshape(MB * L, C)
    xk = xk_ref[...].reshape(MB * L, T)
    enc = jnp.tanh(xm @ Wv_ref[...] + xk @ Wm_ref[...] + be_ref[...])
    uv = enc @ A_s[...]
    u = uv[:, 0:C]
    v = uv[:, C:2 * C]
    for i in range(MB):
        dfb = jnp.mean(enc[i * L:(i + 1) * L], axis=0, keepdims=True)
        df_ref[i] = dfb
        xe = dfb @ Wp_ref[...] + bp_ref[...]
        diff = reg_ref[...] - xe
        d2_ref[i] = jnp.sum(diff * diff, axis=1)[None, :]
        dp = jax.nn.relu(dfb @ Wc1_ref[...] + bc1_ref[...]) @ Wc2_ref[...] + bc2_ref[...]
        dp_ref[i] = dp
        trend = W2_s[...] @ v[i * L:(i + 1) * L]
        s_ref[i] = u[i * L:(i + 1) * L] + trend


_VQ_SC_CACHE = []


def _get_vq_sc():
    if _VQ_SC_CACHE:
        return _VQ_SC_CACHE[0]
    mesh = plsc.VectorSubcoreMesh(core_axis_name="c", subcore_axis_name="s")

    @functools.partial(
        pl.kernel,
        mesh=mesh,
        out_type=[jax.ShapeDtypeStruct((B, 128), jnp.float32),
                  jax.ShapeDtypeStruct((16,), jnp.float32)],
        scratch_types=[pltpu.VMEM((B, R), jnp.float32),
                       pltpu.VMEM((16, 128), jnp.float32),
                       pltpu.VMEM((16,), jnp.float32),
                       pltpu.SemaphoreType.DMA],
    )
    def _vq_sc(d2_hbm, reg3_hbm, cc_hbm, dmin_hbm,
               d2_v, rows_v, mv, sem):
        wid = lax.axis_index("c") * 16 + lax.axis_index("s")

        @pl.when(wid == 0)
        def _():
            pltpu.sync_copy(d2_hbm, d2_v)
            lane = lax.iota(jnp.int32, 16)
            dminv = jnp.zeros((16,), jnp.float32)
            idxv = jnp.zeros((16,), jnp.int32)
            for b in range(B):
                bestv = d2_v[b, pl.ds(0, 16)]
                besti = lane
                for c in range(1, R // 16):
                    v2 = d2_v[b, pl.ds(16 * c, 16)]
                    i2 = lane + 16 * c
                    upd = v2 < bestv
                    bestv = jnp.where(upd, v2, bestv)
                    besti = jnp.where(upd, i2, besti)
                for s in (1, 2, 4, 8):
                    perm = lane ^ s
                    v2 = bestv.at[perm].get(mode="promise_in_bounds")
                    i2 = besti.at[perm].get(mode="promise_in_bounds")
                    upd = (v2 < bestv) | ((v2 == bestv) & (i2 < besti))
                    bestv = jnp.where(upd, v2, bestv)
                    besti = jnp.where(upd, i2, besti)
                sel = lane == b
                dminv = jnp.where(sel, bestv, dminv)
                idxv = jnp.where(sel, besti, idxv)
            mv[...] = dminv
            pltpu.sync_copy(mv, dmin_hbm)
            pltpu.async_copy(reg3_hbm.at[idxv], rows_v, sem).wait()
            pltpu.sync_copy(rows_v, cc_hbm)

    _VQ_SC_CACHE.append(_vq_sc)
    return _vq_sc


def _stage3_body(s_ref, cc_ref, dmin_ref, x_ref, m_ref,
                 w1_ref, b1_ref, w2_ref, b2_ref,
                 out_ref, rf_ref, loss_ref, w1_s, w2_s):
    g = pl.program_id(0)

    @pl.when(g == 0)
    def _init():
        loss_ref[...] = jnp.sum(
            jnp.sqrt(dmin_ref[...]).reshape(1, 16), axis=1, keepdims=True) * (1.0 / B)
        for j in range(3):
            w1_s[j] = jnp.transpose(w1_ref[:, :, j])
            w2_s[j] = jnp.transpose(w2_ref[:, :, j])

    z1 = jnp.zeros((1, C), jnp.float32)
    z2 = jnp.zeros((1, 2 * C), jnp.float32)
    for i in range(MB):
        recon = s_ref[i] + cc_ref[i][:, 0:C]
        rp = jnp.concatenate([z1, recon, z1], axis=0)
        h = rp[0:L] @ w1_s[0] + rp[1:L + 1] @ w1_s[1] + rp[2:L + 2] @ w1_s[2] + b1_ref[...]
        h = jnp.maximum(h, 0.0)
        hp = jnp.concatenate([z2, h, z2], axis=0)
        r2 = hp[0:L] @ w2_s[0] + hp[1:L + 1] @ w2_s[1] + hp[2:L + 2] @ w2_s[2] + b2_ref[...]
        rf_ref[i] = r2
        out_ref[i] = m_ref[i] * x_ref[i] + (1.0 - m_ref[i]) * r2


def _const2(shape):
    return pl.BlockSpec(shape, lambda g: (0, 0))


def _make_stage1():
    f32 = jnp.float32
    return pl.pallas_call(
        _stage1_body,
        grid=(G,),
        in_specs=[
            pl.BlockSpec((MB, L, C), lambda g: (g, 0, 0)),
            pl.BlockSpec((MB, L, C), lambda g: (g, 0, 0)),
            pl.BlockSpec((MB, L, T), lambda g: (g, 0, 0)),
            _const2((C, D)),
            _const2((T, D)),
            _const2((1, D)),
            _const2((R, D)),
            _const2((D, D)),
            _const2((1, D)),
            pl.BlockSpec((D, D), lambda g: (0, 0)),
            pl.BlockSpec((D, D), lambda g: (1, 0)),
            _const2((D, C)),
            _const2((D, C)),
            _const2((1, D)),
            _const2((1, C)),
            _const2((1, C)),
            _const2((D, D // 2)),
            _const2((1, D // 2)),
            _const2((D // 2, ND)),
            _const2((1, ND)),
        ],
        out_specs=[
            pl.BlockSpec((MB, L, C), lambda g: (g, 0, 0)),
            pl.BlockSpec((MB, 1, D), lambda g: (g, 0, 0)),
            pl.BlockSpec((MB, 1, R), lambda g: (g, 0, 0)),
            pl.BlockSpec((MB, 1, ND), lambda g: (g, 0, 0)),
            _const2((R, 128)),
        ],
        out_shape=[
            jax.ShapeDtypeStruct((B, L, C), f32),
            jax.ShapeDtypeStruct((B, 1, D), f32),
            jax.ShapeDtypeStruct((B, 1, R), f32),
            jax.ShapeDtypeStruct((B, 1, ND), f32),
            jax.ShapeDtypeStruct((R, 128), f32),
        ],
        scratch_shapes=[pltpu.VMEM((D, 2 * C), f32),
                        pltpu.VMEM((L, L), f32)],
    )


def _make_stage3():
    f32 = jnp.float32
    return pl.pallas_call(
        _stage3_body,
        grid=(G,),
        in_specs=[
            pl.BlockSpec((MB, L, C), lambda g: (g, 0, 0)),
            pl.BlockSpec((MB, 1, 128), lambda g: (g, 0, 0)),
            pl.BlockSpec((16,), lambda g: (0,)),
            pl.BlockSpec((MB, L, C), lambda g: (g, 0, 0)),
            pl.BlockSpec((MB, L, C), lambda g: (g, 0, 0)),
            pl.BlockSpec((2 * C, C, 3), lambda g: (0, 0, 0)),
            _const2((1, 2 * C)),
            pl.BlockSpec((C, 2 * C, 3), lambda g: (0, 0, 0)),
            _const2((1, C)),
        ],
        out_specs=[
            pl.BlockSpec((MB, L, C), lambda g: (g, 0, 0)),
            pl.BlockSpec((MB, L, C), lambda g: (g, 0, 0)),
            _const2((1, 1)),
        ],
        out_shape=[
            jax.ShapeDtypeStruct((B, L, C), f32),
            jax.ShapeDtypeStruct((B, L, C), f32),
            jax.ShapeDtypeStruct((1, 1), f32),
        ],
        scratch_shapes=[pltpu.VMEM((3, C, 2 * C), f32),
                        pltpu.VMEM((3, 2 * C, C), f32)],
    )


def kernel(x_enc, x_mark_enc, mask, W_val, W_mark, b_enc, register, Wp, bp,
           Wf, bf, Ws, bs, Wt, bt, conv1_w, conv1_b, conv2_w, conv2_b,
           Wc1, bc1, Wc2, bc2):
    be2 = b_enc.reshape(1, D)
    bp2 = bp.reshape(1, D)
    bf2 = bf.reshape(1, D)
    bs2 = bs.reshape(1, C)
    bt2 = bt.reshape(1, C)
    bc1_2 = bc1.reshape(1, D // 2)
    bc2_2 = bc2.reshape(1, ND)
    b1 = conv1_b.reshape(1, 2 * C)
    b2 = conv2_b.reshape(1, C)

    s, df3, d23, dp3, reg3 = _make_stage1()(
        x_enc, mask, x_mark_enc, W_val, W_mark, be2, register, Wp, bp2,
        Wf, Wf, Ws, Wt, bf2, bs2, bt2, Wc1, bc1_2, Wc2, bc2_2)

    cc, dmin = _get_vq_sc()(d23.reshape(B, R), reg3)

    out, refined, loss11 = _make_stage3()(
        s, cc.reshape(B, 1, 128), dmin, x_enc, mask, conv1_w, b1, conv2_w, b2)

    return (out, refined, loss11.reshape(()),
            dp3.reshape(B, ND), df3.reshape(B, D))


# speedup vs baseline: 1.0931x; 1.0931x over previous
import functools

import jax
import jax.numpy as jnp
from jax import lax
from jax.experimental import pallas as pl
from jax.experimental.pallas import tpu as pltpu
from jax.experimental.pallas import tpu_sc as plsc

B, L, C, T = 16, 512, 32, 4
D, R, NR, ND, K = 512, 128, 3, 3, 25
MB = 4
G = B // MB

_AT_B = (((0,), (0,)), ((), ()))
_HI = lax.Precision.HIGHEST


def _atb(a, b):
    return lax.dot_general(a, b, dimension_numbers=_AT_B,
                           preferred_element_type=jnp.float32)


def _stage1_body(x_ref, m_ref, xk_ref, Wv_ref, Wm_ref, be_ref,
                 regT_ref, reg_ref, Wp_ref, bp_ref,
                 Wft_ref, Wfb_ref, Ws_ref, Wt_ref, bf_ref, bs_ref, bt_ref,
                 Wc1_ref, bc1_ref, Wc2_ref, bc2_ref,
                 s_ref, df_ref, d2_ref, dp_ref, reg3_ref,
                 A_s, W2T_s, df_s, d2_s, dp_s):
    g = pl.program_id(0)

    @pl.when(g == 0)
    def _fold():
        Ws = Ws_ref[...]
        Wt = Wt_ref[...]
        A_s[...] = jnp.dot(Wft_ref[...], jnp.concatenate([Ws, Wt - Ws], axis=1),
                           precision=_HI, preferred_element_type=jnp.float32)
        Qm = jnp.dot(Wfb_ref[...], Wt, precision=_HI,
                     preferred_element_type=jnp.float32)
        reg3 = jnp.dot(reg_ref[...], Qm, precision=_HI,
                       preferred_element_type=jnp.float32)
        reg3 = reg3 + (bf_ref[...] @ Wt + bs_ref[...] + bt_ref[...])
        reg3_ref[...] = jnp.concatenate(
            [reg3, jnp.zeros((R, 128 - C), jnp.float32)], axis=1)
        ri = lax.broadcasted_iota(jnp.int32, (L, L), 0)
        ci = lax.broadcasted_iota(jnp.int32, (L, L), 1)
        band = ((ci >= ri - 12) & (ci <= ri + 12)).astype(jnp.float32)
        ex0 = jnp.where(ri == 0, jnp.maximum(12 - ci, 0), 0).astype(jnp.float32)
        ex1 = jnp.where(ri == L - 1, jnp.maximum(ci - (L - 13), 0), 0).astype(jnp.float32)
        W2T_s[...] = (band + ex0 + ex1) * (1.0 / K)

    ones_col = jnp.full((L, 1), 1.0 / L, jnp.float32)
    for i in range(MB):
        xmT = x_ref[i] * m_ref[i]
        encT = jnp.tanh(_atb(Wv_ref[...], xmT) + _atb(Wm_ref[...], xk_ref[i])
                        + be_ref[...])
        df_col = jnp.mean(encT, axis=1, keepdims=True)
        df_row = lax.dot_general(ones_col, encT, (((0,), (1,)), ((), ())),
                                 preferred_element_type=jnp.float32)
        df_s[pl.ds(g * MB + i, 1)] = df_row
        uvT = _atb(A_s[...], encT)
        trendT = uvT[C:2 * C, :] @ W2T_s[...]
        s_ref[i] = uvT[0:C, :] + trendT
        xe_col = _atb(Wp_ref[...], df_col) + bp_ref[...]
        diffT = regT_ref[...] - xe_col
        d2_s[pl.ds(g * MB + i, 1)] = jnp.sum(diffT * diffT, axis=0)[None, :]
        h1 = jax.nn.relu(df_row @ Wc1_ref[...] + bc1_ref[...])
        dp_s[pl.ds(g * MB + i, 1)] = h1 @ Wc2_ref[...] + bc2_ref[...]

    @pl.when(g == G - 1)
    def _flush():
        df_ref[...] = df_s[...]
        d2_ref[...] = d2_s[...]
        dp_ref[...] = dp_s[...]


_VQ_SC_CACHE = []


def _get_vq_sc():
    if _VQ_SC_CACHE:
        return _VQ_SC_CACHE[0]
    mesh = plsc.VectorSubcoreMesh(core_axis_name="c", subcore_axis_name="s")

    @functools.partial(
        pl.kernel,
        mesh=mesh,
        out_type=[jax.ShapeDtypeStruct((B, 128), jnp.float32),
                  jax.ShapeDtypeStruct((16,), jnp.float32)],
        scratch_types=[pltpu.VMEM((B, R), jnp.float32),
                       pltpu.VMEM((16, 128), jnp.float32),
                       pltpu.VMEM((16,), jnp.float32),
                       pltpu.SemaphoreType.DMA],
    )
    def _vq_sc(d2_hbm, reg3_hbm, cc_hbm, dmin_hbm,
               d2_v, rows_v, mv, sem):
        wid = lax.axis_index("c") * 16 + lax.axis_index("s")

        @pl.when(wid == 0)
        def _():
            pltpu.sync_copy(d2_hbm, d2_v)
            lane = lax.iota(jnp.int32, 16)
            dminv = jnp.zeros((16,), jnp.float32)
            idxv = jnp.zeros((16,), jnp.int32)
            for b in range(B):
                bestv = d2_v[b, pl.ds(0, 16)]
                besti = lane
                for c in range(1, R // 16):
                    v2 = d2_v[b, pl.ds(16 * c, 16)]
                    i2 = lane + 16 * c
                    upd = v2 < bestv
                    bestv = jnp.where(upd, v2, bestv)
                    besti = jnp.where(upd, i2, besti)
                for s in (1, 2, 4, 8):
                    perm = lane ^ s
                    v2 = bestv.at[perm].get(mode="promise_in_bounds")
                    i2 = besti.at[perm].get(mode="promise_in_bounds")
                    upd = (v2 < bestv) | ((v2 == bestv) & (i2 < besti))
                    bestv = jnp.where(upd, v2, bestv)
                    besti = jnp.where(upd, i2, besti)
                sel = lane == b
                dminv = jnp.where(sel, bestv, dminv)
                idxv = jnp.where(sel, besti, idxv)
            mv[...] = dminv
            pltpu.sync_copy(mv, dmin_hbm)
            pltpu.async_copy(reg3_hbm.at[idxv], rows_v, sem).wait()
            pltpu.sync_copy(rows_v, cc_hbm)

    _VQ_SC_CACHE.append(_vq_sc)
    return _vq_sc


def _stage3_body(s_ref, cc_ref, dmin_ref, x_ref, m_ref,
                 w1_ref, b1_ref, w2_ref, b2_ref,
                 out_ref, rf_ref, loss_ref, w1_s, w2_s):
    g = pl.program_id(0)

    @pl.when(g == 0)
    def _init():
        loss_ref[...] = jnp.sum(
            jnp.sqrt(dmin_ref[...]).reshape(1, 16), axis=1, keepdims=True) * (1.0 / B)
        for j in range(3):
            w1_s[j] = w1_ref[:, :, j]
            w2_s[j] = w2_ref[:, :, j]

    z1 = jnp.zeros((C, 1), jnp.float32)
    z2 = jnp.zeros((2 * C, 1), jnp.float32)
    for i in range(MB):
        ccT = jnp.transpose(cc_ref[i])
        reconT = s_ref[i] + ccT[0:C, :]
        rp = jnp.concatenate([z1, reconT, z1], axis=1)
        h = (w1_s[0] @ rp[:, 0:L] + w1_s[1] @ rp[:, 1:L + 1]
             + w1_s[2] @ rp[:, 2:L + 2] + b1_ref[...])
        h = jnp.maximum(h, 0.0)
        hp = jnp.concatenate([z2, h, z2], axis=1)
        r2 = (w2_s[0] @ hp[:, 0:L] + w2_s[1] @ hp[:, 1:L + 1]
              + w2_s[2] @ hp[:, 2:L + 2] + b2_ref[...])
        rf_ref[i] = r2
        out_ref[i] = m_ref[i] * x_ref[i] + (1.0 - m_ref[i]) * r2


def _const2(shape):
    return pl.BlockSpec(shape, lambda g: (0, 0))


def _make_stage1():
    f32 = jnp.float32
    return pl.pallas_call(
        _stage1_body,
        grid=(G,),
        in_specs=[
            pl.BlockSpec((MB, C, L), lambda g: (g, 0, 0)),
            pl.BlockSpec((MB, C, L), lambda g: (g, 0, 0)),
            pl.BlockSpec((MB, T, L), lambda g: (g, 0, 0)),
            _const2((C, D)),
            _const2((T, D)),
            _const2((D, 1)),
            _const2((D, R)),
            _const2((R, D)),
            _const2((D, D)),
            _const2((D, 1)),
            pl.BlockSpec((D, D), lambda g: (0, 0)),
            pl.BlockSpec((D, D), lambda g: (1, 0)),
            _const2((D, C)),
            _const2((D, C)),
            _const2((1, D)),
            _const2((1, C)),
            _const2((1, C)),
            _const2((D, D // 2)),
            _const2((1, D // 2)),
            _const2((D // 2, ND)),
            _const2((1, ND)),
        ],
        out_specs=[
            pl.BlockSpec((MB, C, L), lambda g: (g, 0, 0)),
            _const2((B, D)),
            _const2((B, R)),
            _const2((B, ND)),
            _const2((R, 128)),
        ],
        out_shape=[
            jax.ShapeDtypeStruct((B, C, L), f32),
            jax.ShapeDtypeStruct((B, D), f32),
            jax.ShapeDtypeStruct((B, R), f32),
            jax.ShapeDtypeStruct((B, ND), f32),
            jax.ShapeDtypeStruct((R, 128), f32),
        ],
        scratch_shapes=[pltpu.VMEM((D, 2 * C), f32),
                        pltpu.VMEM((L, L), f32),
                        pltpu.VMEM((B, D), f32),
                        pltpu.VMEM((B, R), f32),
                        pltpu.VMEM((B, ND), f32)],
    )


def _make_stage3():
    f32 = jnp.float32
    return pl.pallas_call(
        _stage3_body,
        grid=(G,),
        in_specs=[
            pl.BlockSpec((MB, C, L), lambda g: (g, 0, 0)),
            pl.BlockSpec((MB, 1, 128), lambda g: (g, 0, 0)),
            pl.BlockSpec((16,), lambda g: (0,)),
            pl.BlockSpec((MB, C, L), lambda g: (g, 0, 0)),
            pl.BlockSpec((MB, C, L), lambda g: (g, 0, 0)),
            pl.BlockSpec((2 * C, C, 3), lambda g: (0, 0, 0)),
            _const2((2 * C, 1)),
            pl.BlockSpec((C, 2 * C, 3), lambda g: (0, 0, 0)),
            _const2((C, 1)),
        ],
        out_specs=[
            pl.BlockSpec((MB, C, L), lambda g: (g, 0, 0)),
            pl.BlockSpec((MB, C, L), lambda g: (g, 0, 0)),
            _const2((1, 1)),
        ],
        out_shape=[
            jax.ShapeDtypeStruct((B, C, L), f32),
            jax.ShapeDtypeStruct((B, C, L), f32),
            jax.ShapeDtypeStruct((1, 1), f32),
        ],
        scratch_shapes=[pltpu.VMEM((3, 2 * C, C), f32),
                        pltpu.VMEM((3, C, 2 * C), f32)],
    )


def kernel(x_enc, x_mark_enc, mask, W_val, W_mark, b_enc, register, Wp, bp,
           Wf, bf, Ws, bs, Wt, bt, conv1_w, conv1_b, conv2_w, conv2_b,
           Wc1, bc1, Wc2, bc2):
    xT = jnp.transpose(x_enc, (0, 2, 1))
    mT = jnp.transpose(mask, (0, 2, 1))
    kT = jnp.transpose(x_mark_enc, (0, 2, 1))
    regT = jnp.transpose(register)
    be2 = b_enc.reshape(D, 1)
    bp2 = bp.reshape(D, 1)
    bf2 = bf.reshape(1, D)
    bs2 = bs.reshape(1, C)
    bt2 = bt.reshape(1, C)
    bc1_2 = bc1.reshape(1, D // 2)
    bc2_2 = bc2.reshape(1, ND)
    b1 = conv1_b.reshape(2 * C, 1)
    b2 = conv2_b.reshape(C, 1)

    sT, df, d2, dp, reg3 = _make_stage1()(
        xT, mT, kT, W_val, W_mark, be2, regT, register, Wp, bp2,
        Wf, Wf, Ws, Wt, bf2, bs2, bt2, Wc1, bc1_2, Wc2, bc2_2)

    cc, dmin = _get_vq_sc()(d2, reg3)

    outT, refT, loss11 = _make_stage3()(
        sT, cc.reshape(B, 1, 128), dmin, xT, mT, conv1_w, b1, conv2_w, b2)

    return (jnp.transpose(outT, (0, 2, 1)), jnp.transpose(refT, (0, 2, 1)),
            loss11.reshape(()), dp, df)


# speedup vs baseline: 1.0948x; 1.0016x over previous
import functools

import jax
import jax.numpy as jnp
from jax import lax
from jax.experimental import pallas as pl
from jax.experimental.pallas import tpu as pltpu
from jax.experimental.pallas import tpu_sc as plsc

B, L, C, T = 16, 512, 32, 4
D, R, NR, ND, K = 512, 128, 3, 3, 25
MB = 16
G = B // MB

_AT_B = (((0,), (0,)), ((), ()))
_HI = lax.Precision.HIGHEST


def _atb(a, b):
    return lax.dot_general(a, b, dimension_numbers=_AT_B,
                           preferred_element_type=jnp.float32)


def _stage1_body(x_ref, m_ref, xk_ref, Wv_ref, Wm_ref, be_ref,
                 regT_ref, reg_ref, Wp_ref, bp_ref,
                 Wft_ref, Wfb_ref, Ws_ref, Wt_ref, bf_ref, bs_ref, bt_ref,
                 Wc1_ref, bc1_ref, Wc2_ref, bc2_ref,
                 s_ref, df_ref, d2_ref, dp_ref, reg3_ref,
                 A_s, W2T_s, df_s, d2_s, dp_s):
    g = pl.program_id(0)

    @pl.when(g == 0)
    def _fold():
        Ws = Ws_ref[...]
        Wt = Wt_ref[...]
        A_s[...] = Wft_ref[...] @ jnp.concatenate([Ws, Wt - Ws], axis=1)
        Qm = Wfb_ref[...] @ Wt
        reg3 = reg_ref[...] @ Qm + (bf_ref[...] @ Wt + bs_ref[...] + bt_ref[...])
        reg3_ref[...] = jnp.concatenate(
            [reg3, jnp.zeros((R, 128 - C), jnp.float32)], axis=1)
        ri = lax.broadcasted_iota(jnp.int32, (L, L), 0)
        ci = lax.broadcasted_iota(jnp.int32, (L, L), 1)
        band = ((ci >= ri - 12) & (ci <= ri + 12)).astype(jnp.float32)
        ex0 = jnp.where(ri == 0, jnp.maximum(12 - ci, 0), 0).astype(jnp.float32)
        ex1 = jnp.where(ri == L - 1, jnp.maximum(ci - (L - 13), 0), 0).astype(jnp.float32)
        W2T_s[...] = (band + ex0 + ex1) * (1.0 / K)

    ones_col = jnp.full((L, 1), 1.0 / L, jnp.float32)
    for i in range(MB):
        xmT = x_ref[i] * m_ref[i]
        encT = jnp.tanh(_atb(Wv_ref[...], xmT) + _atb(Wm_ref[...], xk_ref[i])
                        + be_ref[...])
        df_col = jnp.mean(encT, axis=1, keepdims=True)
        df_row = lax.dot_general(ones_col, encT, (((0,), (1,)), ((), ())),
                                 preferred_element_type=jnp.float32)
        df_s[pl.ds(g * MB + i, 1)] = df_row
        uvT = _atb(A_s[...], encT)
        trendT = uvT[C:2 * C, :] @ W2T_s[...]
        s_ref[i] = uvT[0:C, :] + trendT
        xe_col = _atb(Wp_ref[...], df_col) + bp_ref[...]
        diffT = regT_ref[...] - xe_col
        d2_s[pl.ds(g * MB + i, 1)] = jnp.sum(diffT * diffT, axis=0)[None, :]
        h1 = jax.nn.relu(df_row @ Wc1_ref[...] + bc1_ref[...])
        dp_s[pl.ds(g * MB + i, 1)] = h1 @ Wc2_ref[...] + bc2_ref[...]

    @pl.when(g == G - 1)
    def _flush():
        df_ref[...] = df_s[...]
        d2_ref[...] = d2_s[...]
        dp_ref[...] = dp_s[...]


_VQ_SC_CACHE = []


def _get_vq_sc():
    if _VQ_SC_CACHE:
        return _VQ_SC_CACHE[0]
    mesh = plsc.VectorSubcoreMesh(core_axis_name="c", subcore_axis_name="s")

    @functools.partial(
        pl.kernel,
        mesh=mesh,
        out_type=[jax.ShapeDtypeStruct((B, 128), jnp.float32),
                  jax.ShapeDtypeStruct((16,), jnp.float32)],
        scratch_types=[pltpu.VMEM((B, R), jnp.float32),
                       pltpu.VMEM((16, 128), jnp.float32),
                       pltpu.VMEM((16,), jnp.float32),
                       pltpu.SemaphoreType.DMA],
    )
    def _vq_sc(d2_hbm, reg3_hbm, cc_hbm, dmin_hbm,
               d2_v, rows_v, mv, sem):
        wid = lax.axis_index("c") * 16 + lax.axis_index("s")

        @pl.when(wid == 0)
        def _():
            pltpu.sync_copy(d2_hbm, d2_v)
            lane = lax.iota(jnp.int32, 16)
            dminv = jnp.zeros((16,), jnp.float32)
            idxv = jnp.zeros((16,), jnp.int32)
            for b in range(B):
                bestv = d2_v[b, pl.ds(0, 16)]
                besti = lane
                for c in range(1, R // 16):
                    v2 = d2_v[b, pl.ds(16 * c, 16)]
                    i2 = lane + 16 * c
                    upd = v2 < bestv
                    bestv = jnp.where(upd, v2, bestv)
                    besti = jnp.where(upd, i2, besti)
                for s in (1, 2, 4, 8):
                    perm = lane ^ s
                    v2 = bestv.at[perm].get(mode="promise_in_bounds")
                    i2 = besti.at[perm].get(mode="promise_in_bounds")
                    upd = (v2 < bestv) | ((v2 == bestv) & (i2 < besti))
                    bestv = jnp.where(upd, v2, bestv)
                    besti = jnp.where(upd, i2, besti)
                sel = lane == b
                dminv = jnp.where(sel, bestv, dminv)
                idxv = jnp.where(sel, besti, idxv)
            mv[...] = dminv
            pltpu.sync_copy(mv, dmin_hbm)
            pltpu.async_copy(reg3_hbm.at[idxv], rows_v, sem).wait()
            pltpu.sync_copy(rows_v, cc_hbm)

    _VQ_SC_CACHE.append(_vq_sc)
    return _vq_sc


def _stage3_body(s_ref, cc_ref, dmin_ref, x_ref, m_ref,
                 w1_ref, b1_ref, w2_ref, b2_ref,
                 out_ref, rf_ref, loss_ref, w1_s, w2_s):
    g = pl.program_id(0)

    @pl.when(g == 0)
    def _init():
        loss_ref[...] = jnp.sum(
            jnp.sqrt(dmin_ref[...]).reshape(1, 16), axis=1, keepdims=True) * (1.0 / B)
        for j in range(3):
            w1_s[j] = w1_ref[:, :, j]
            w2_s[j] = w2_ref[:, :, j]

    z1 = jnp.zeros((C, 1), jnp.float32)
    z2 = jnp.zeros((2 * C, 1), jnp.float32)
    for i in range(MB):
        ccT = jnp.transpose(cc_ref[i])
        reconT = s_ref[i] + ccT[0:C, :]
        rp = jnp.concatenate([z1, reconT, z1], axis=1)
        h = (w1_s[0] @ rp[:, 0:L] + w1_s[1] @ rp[:, 1:L + 1]
             + w1_s[2] @ rp[:, 2:L + 2] + b1_ref[...])
        h = jnp.maximum(h, 0.0)
        hp = jnp.concatenate([z2, h, z2], axis=1)
        r2 = (w2_s[0] @ hp[:, 0:L] + w2_s[1] @ hp[:, 1:L + 1]
              + w2_s[2] @ hp[:, 2:L + 2] + b2_ref[...])
        rf_ref[i] = r2
        out_ref[i] = m_ref[i] * x_ref[i] + (1.0 - m_ref[i]) * r2


def _const2(shape):
    return pl.BlockSpec(shape, lambda g: (0, 0))


def _make_stage1():
    f32 = jnp.float32
    return pl.pallas_call(
        _stage1_body,
        grid=(G,),
        in_specs=[
            pl.BlockSpec((MB, C, L), lambda g: (g, 0, 0)),
            pl.BlockSpec((MB, C, L), lambda g: (g, 0, 0)),
            pl.BlockSpec((MB, T, L), lambda g: (g, 0, 0)),
            _const2((C, D)),
            _const2((T, D)),
            _const2((D, 1)),
            _const2((D, R)),
            _const2((R, D)),
            _const2((D, D)),
            _const2((D, 1)),
            pl.BlockSpec((D, D), lambda g: (0, 0)),
            pl.BlockSpec((D, D), lambda g: (1, 0)),
            _const2((D, C)),
            _const2((D, C)),
            _const2((1, D)),
            _const2((1, C)),
            _const2((1, C)),
            _const2((D, D // 2)),
            _const2((1, D // 2)),
            _const2((D // 2, ND)),
            _const2((1, ND)),
        ],
        out_specs=[
            pl.BlockSpec((MB, C, L), lambda g: (g, 0, 0)),
            _const2((B, D)),
            _const2((B, R)),
            _const2((B, ND)),
            _const2((R, 128)),
        ],
        out_shape=[
            jax.ShapeDtypeStruct((B, C, L), f32),
            jax.ShapeDtypeStruct((B, D), f32),
            jax.ShapeDtypeStruct((B, R), f32),
            jax.ShapeDtypeStruct((B, ND), f32),
            jax.ShapeDtypeStruct((R, 128), f32),
        ],
        scratch_shapes=[pltpu.VMEM((D, 2 * C), f32),
                        pltpu.VMEM((L, L), f32),
                        pltpu.VMEM((B, D), f32),
                        pltpu.VMEM((B, R), f32),
                        pltpu.VMEM((B, ND), f32)],
    )


def _make_stage3():
    f32 = jnp.float32
    return pl.pallas_call(
        _stage3_body,
        grid=(G,),
        in_specs=[
            pl.BlockSpec((MB, C, L), lambda g: (g, 0, 0)),
            pl.BlockSpec((MB, 1, 128), lambda g: (g, 0, 0)),
            pl.BlockSpec((16,), lambda g: (0,)),
            pl.BlockSpec((MB, C, L), lambda g: (g, 0, 0)),
            pl.BlockSpec((MB, C, L), lambda g: (g, 0, 0)),
            pl.BlockSpec((2 * C, C, 3), lambda g: (0, 0, 0)),
            _const2((2 * C, 1)),
            pl.BlockSpec((C, 2 * C, 3), lambda g: (0, 0, 0)),
            _const2((C, 1)),
        ],
        out_specs=[
            pl.BlockSpec((MB, C, L), lambda g: (g, 0, 0)),
            pl.BlockSpec((MB, C, L), lambda g: (g, 0, 0)),
            _const2((1, 1)),
        ],
        out_shape=[
            jax.ShapeDtypeStruct((B, C, L), f32),
            jax.ShapeDtypeStruct((B, C, L), f32),
            jax.ShapeDtypeStruct((1, 1), f32),
        ],
        scratch_shapes=[pltpu.VMEM((3, 2 * C, C), f32),
                        pltpu.VMEM((3, C, 2 * C), f32)],
    )


def kernel(x_enc, x_mark_enc, mask, W_val, W_mark, b_enc, register, Wp, bp,
           Wf, bf, Ws, bs, Wt, bt, conv1_w, conv1_b, conv2_w, conv2_b,
           Wc1, bc1, Wc2, bc2):
    xT = jnp.transpose(x_enc, (0, 2, 1))
    mT = jnp.transpose(mask, (0, 2, 1))
    kT = jnp.transpose(x_mark_enc, (0, 2, 1))
    regT = jnp.transpose(register)
    be2 = b_enc.reshape(D, 1)
    bp2 = bp.reshape(D, 1)
    bf2 = bf.reshape(1, D)
    bs2 = bs.reshape(1, C)
    bt2 = bt.reshape(1, C)
    bc1_2 = bc1.reshape(1, D // 2)
    bc2_2 = bc2.reshape(1, ND)
    b1 = conv1_b.reshape(2 * C, 1)
    b2 = conv2_b.reshape(C, 1)

    sT, df, d2, dp, reg3 = _make_stage1()(
        xT, mT, kT, W_val, W_mark, be2, regT, register, Wp, bp2,
        Wf, Wf, Ws, Wt, bf2, bs2, bt2, Wc1, bc1_2, Wc2, bc2_2)

    cc, dmin = _get_vq_sc()(d2, reg3)

    outT, refT, loss11 = _make_stage3()(
        sT, cc.reshape(B, 1, 128), dmin, xT, mT, conv1_w, b1, conv2_w, b2)

    return (jnp.transpose(outT, (0, 2, 1)), jnp.transpose(refT, (0, 2, 1)),
            loss11.reshape(()), dp, df)


# speedup vs baseline: 1.1162x; 1.0195x over previous
import functools

import jax
import jax.numpy as jnp
from jax import lax
from jax.experimental import pallas as pl
from jax.experimental.pallas import tpu as pltpu
from jax.experimental.pallas import tpu_sc as plsc

B, L, C, T = 16, 512, 32, 4
D, R, NR, ND, K = 512, 128, 3, 3, 25
MB = 16
G = B // MB

_AT_B = (((0,), (0,)), ((), ()))
_HI = lax.Precision.HIGHEST


def _atb(a, b):
    return lax.dot_general(a, b, dimension_numbers=_AT_B,
                           preferred_element_type=jnp.float32)


def _stage1_body(x_ref, m_ref, xk_ref, Wv_ref, Wm_ref, be_ref,
                 regT_ref, reg_ref, Wp_ref, bp_ref,
                 Wft_ref, Wfb_ref, Ws_ref, Wt_ref, bf_ref, bs_ref, bt_ref,
                 Wc1_ref, bc1_ref, Wc2_ref, bc2_ref,
                 s_ref, df_ref, d2_ref, dp_ref, reg3_ref,
                 A_s, W2T_s, df_s, d2_s, dp_s):
    g = pl.program_id(0)

    @pl.when(g == 0)
    def _fold():
        Ws = Ws_ref[...]
        Wt = Wt_ref[...]
        A_s[...] = Wft_ref[...] @ jnp.concatenate([Ws, Wt - Ws], axis=1)
        Qm = Wfb_ref[...] @ Wt
        reg3 = reg_ref[...] @ Qm + (bf_ref[...] @ Wt + bs_ref[...] + bt_ref[...])
        reg3_ref[...] = jnp.concatenate(
            [reg3, jnp.zeros((R, 128 - C), jnp.float32)], axis=1)
        ri = lax.broadcasted_iota(jnp.int32, (L, L), 0)
        ci = lax.broadcasted_iota(jnp.int32, (L, L), 1)
        band = ((ci >= ri - 12) & (ci <= ri + 12)).astype(jnp.float32)
        ex0 = jnp.where(ri == 0, jnp.maximum(12 - ci, 0), 0).astype(jnp.float32)
        ex1 = jnp.where(ri == L - 1, jnp.maximum(ci - (L - 13), 0), 0).astype(jnp.float32)
        W2T_s[...] = (band + ex0 + ex1) * (1.0 / K)

    ones_col = jnp.full((L, 1), 1.0 / L, jnp.float32)
    for i in range(MB):
        xmT = x_ref[i] * m_ref[i]
        encT = jnp.tanh(_atb(Wv_ref[...], xmT) + _atb(Wm_ref[...], xk_ref[i])
                        + be_ref[...])
        df_col = jnp.mean(encT, axis=1, keepdims=True)
        df_row = lax.dot_general(ones_col, encT, (((0,), (1,)), ((), ())),
                                 preferred_element_type=jnp.float32)
        df_s[pl.ds(g * MB + i, 1)] = df_row
        uvT = _atb(A_s[...], encT)
        trendT = uvT[C:2 * C, :] @ W2T_s[...]
        s_ref[i] = uvT[0:C, :] + trendT
        xe_col = _atb(Wp_ref[...], df_col) + bp_ref[...]
        diffT = regT_ref[...] - xe_col
        d2_s[pl.ds(g * MB + i, 1)] = jnp.sum(diffT * diffT, axis=0)[None, :]
        h1 = jax.nn.relu(df_row @ Wc1_ref[...] + bc1_ref[...])
        dp_s[pl.ds(g * MB + i, 1)] = h1 @ Wc2_ref[...] + bc2_ref[...]

    @pl.when(g == G - 1)
    def _flush():
        df_ref[...] = df_s[...]
        d2_ref[...] = d2_s[...]
        dp_ref[...] = dp_s[...]


_VQ_SC_CACHE = []


def _get_vq_sc():
    if _VQ_SC_CACHE:
        return _VQ_SC_CACHE[0]
    mesh = plsc.VectorSubcoreMesh(core_axis_name="c", subcore_axis_name="s",
                                  num_cores=1)

    @functools.partial(
        pl.kernel,
        mesh=mesh,
        out_type=[jax.ShapeDtypeStruct((B, 128), jnp.float32),
                  jax.ShapeDtypeStruct((16,), jnp.float32)],
        scratch_types=[pltpu.VMEM((B, R), jnp.float32),
                       pltpu.VMEM((16, 128), jnp.float32),
                       pltpu.VMEM((16,), jnp.float32),
                       pltpu.SemaphoreType.DMA],
    )
    def _vq_sc(d2_hbm, reg3_hbm, cc_hbm, dmin_hbm,
               d2_v, rows_v, mv, sem):
        wid = lax.axis_index("c") * 16 + lax.axis_index("s")

        @pl.when(wid == 0)
        def _():
            pltpu.sync_copy(d2_hbm, d2_v)
            lane = lax.iota(jnp.int32, 16)
            dminv = jnp.zeros((16,), jnp.float32)
            idxv = jnp.zeros((16,), jnp.int32)
            for b in range(B):
                bestv = d2_v[b, pl.ds(0, 16)]
                besti = lane
                for c in range(1, R // 16):
                    v2 = d2_v[b, pl.ds(16 * c, 16)]
                    i2 = lane + 16 * c
                    upd = v2 < bestv
                    bestv = jnp.where(upd, v2, bestv)
                    besti = jnp.where(upd, i2, besti)
                for s in (1, 2, 4, 8):
                    perm = lane ^ s
                    v2 = bestv.at[perm].get(mode="promise_in_bounds")
                    i2 = besti.at[perm].get(mode="promise_in_bounds")
                    upd = (v2 < bestv) | ((v2 == bestv) & (i2 < besti))
                    bestv = jnp.where(upd, v2, bestv)
                    besti = jnp.where(upd, i2, besti)
                sel = lane == b
                dminv = jnp.where(sel, bestv, dminv)
                idxv = jnp.where(sel, besti, idxv)
            mv[...] = dminv
            pltpu.sync_copy(mv, dmin_hbm)
            pltpu.async_copy(reg3_hbm.at[idxv], rows_v, sem).wait()
            pltpu.sync_copy(rows_v, cc_hbm)

    _VQ_SC_CACHE.append(_vq_sc)
    return _vq_sc


def _stage3_body(s_ref, cc_ref, dmin_ref, x_ref, m_ref,
                 w1_ref, b1_ref, w2_ref, b2_ref,
                 out_ref, rf_ref, loss_ref, w1_s, w2_s):
    g = pl.program_id(0)

    @pl.when(g == 0)
    def _init():
        loss_ref[...] = jnp.sum(
            jnp.sqrt(dmin_ref[...]).reshape(1, 16), axis=1, keepdims=True) * (1.0 / B)
        for j in range(3):
            w1_s[j] = w1_ref[:, :, j]
            w2_s[j] = w2_ref[:, :, j]

    z1 = jnp.zeros((C, 1), jnp.float32)
    z2 = jnp.zeros((2 * C, 1), jnp.float32)
    for i in range(MB):
        ccT = jnp.transpose(cc_ref[i])
        reconT = s_ref[i] + ccT[0:C, :]
        rp = jnp.concatenate([z1, reconT, z1], axis=1)
        h = (w1_s[0] @ rp[:, 0:L] + w1_s[1] @ rp[:, 1:L + 1]
             + w1_s[2] @ rp[:, 2:L + 2] + b1_ref[...])
        h = jnp.maximum(h, 0.0)
        hp = jnp.concatenate([z2, h, z2], axis=1)
        r2 = (w2_s[0] @ hp[:, 0:L] + w2_s[1] @ hp[:, 1:L + 1]
              + w2_s[2] @ hp[:, 2:L + 2] + b2_ref[...])
        rf_ref[i] = r2
        out_ref[i] = m_ref[i] * x_ref[i] + (1.0 - m_ref[i]) * r2


def _const2(shape):
    return pl.BlockSpec(shape, lambda g: (0, 0))


def _make_stage1():
    f32 = jnp.float32
    return pl.pallas_call(
        _stage1_body,
        grid=(G,),
        in_specs=[
            pl.BlockSpec((MB, C, L), lambda g: (g, 0, 0)),
            pl.BlockSpec((MB, C, L), lambda g: (g, 0, 0)),
            pl.BlockSpec((MB, T, L), lambda g: (g, 0, 0)),
            _const2((C, D)),
            _const2((T, D)),
            _const2((D, 1)),
            _const2((D, R)),
            _const2((R, D)),
            _const2((D, D)),
            _const2((D, 1)),
            pl.BlockSpec((D, D), lambda g: (0, 0)),
            pl.BlockSpec((D, D), lambda g: (1, 0)),
            _const2((D, C)),
            _const2((D, C)),
            _const2((1, D)),
            _const2((1, C)),
            _const2((1, C)),
            _const2((D, D // 2)),
            _const2((1, D // 2)),
            _const2((D // 2, ND)),
            _const2((1, ND)),
        ],
        out_specs=[
            pl.BlockSpec((MB, C, L), lambda g: (g, 0, 0)),
            _const2((B, D)),
            _const2((B, R)),
            _const2((B, ND)),
            _const2((R, 128)),
        ],
        out_shape=[
            jax.ShapeDtypeStruct((B, C, L), f32),
            jax.ShapeDtypeStruct((B, D), f32),
            jax.ShapeDtypeStruct((B, R), f32),
            jax.ShapeDtypeStruct((B, ND), f32),
            jax.ShapeDtypeStruct((R, 128), f32),
        ],
        scratch_shapes=[pltpu.VMEM((D, 2 * C), f32),
                        pltpu.VMEM((L, L), f32),
                        pltpu.VMEM((B, D), f32),
                        pltpu.VMEM((B, R), f32),
                        pltpu.VMEM((B, ND), f32)],
    )


def _make_stage3():
    f32 = jnp.float32
    return pl.pallas_call(
        _stage3_body,
        grid=(G,),
        in_specs=[
            pl.BlockSpec((MB, C, L), lambda g: (g, 0, 0)),
            pl.BlockSpec((MB, 1, 128), lambda g: (g, 0, 0)),
            pl.BlockSpec((16,), lambda g: (0,)),
            pl.BlockSpec((MB, C, L), lambda g: (g, 0, 0)),
            pl.BlockSpec((MB, C, L), lambda g: (g, 0, 0)),
            pl.BlockSpec((2 * C, C, 3), lambda g: (0, 0, 0)),
            _const2((2 * C, 1)),
            pl.BlockSpec((C, 2 * C, 3), lambda g: (0, 0, 0)),
            _const2((C, 1)),
        ],
        out_specs=[
            pl.BlockSpec((MB, C, L), lambda g: (g, 0, 0)),
            pl.BlockSpec((MB, C, L), lambda g: (g, 0, 0)),
            _const2((1, 1)),
        ],
        out_shape=[
            jax.ShapeDtypeStruct((B, C, L), f32),
            jax.ShapeDtypeStruct((B, C, L), f32),
            jax.ShapeDtypeStruct((1, 1), f32),
        ],
        scratch_shapes=[pltpu.VMEM((3, 2 * C, C), f32),
                        pltpu.VMEM((3, C, 2 * C), f32)],
    )


def kernel(x_enc, x_mark_enc, mask, W_val, W_mark, b_enc, register, Wp, bp,
           Wf, bf, Ws, bs, Wt, bt, conv1_w, conv1_b, conv2_w, conv2_b,
           Wc1, bc1, Wc2, bc2):
    xT = jnp.transpose(x_enc, (0, 2, 1))
    mT = jnp.transpose(mask, (0, 2, 1))
    kT = jnp.transpose(x_mark_enc, (0, 2, 1))
    regT = jnp.transpose(register)
    be2 = b_enc.reshape(D, 1)
    bp2 = bp.reshape(D, 1)
    bf2 = bf.reshape(1, D)
    bs2 = bs.reshape(1, C)
    bt2 = bt.reshape(1, C)
    bc1_2 = bc1.reshape(1, D // 2)
    bc2_2 = bc2.reshape(1, ND)
    b1 = conv1_b.reshape(2 * C, 1)
    b2 = conv2_b.reshape(C, 1)

    sT, df, d2, dp, reg3 = _make_stage1()(
        xT, mT, kT, W_val, W_mark, be2, regT, register, Wp, bp2,
        Wf, Wf, Ws, Wt, bf2, bs2, bt2, Wc1, bc1_2, Wc2, bc2_2)

    cc, dmin = _get_vq_sc()(d2, reg3)

    outT, refT, loss11 = _make_stage3()(
        sT, cc.reshape(B, 1, 128), dmin, xT, mT, conv1_w, b1, conv2_w, b2)

    return (jnp.transpose(outT, (0, 2, 1)), jnp.transpose(refT, (0, 2, 1)),
            loss11.reshape(()), dp, df)


# speedup vs baseline: 1.1166x; 1.0003x over previous
import functools

import jax
import jax.numpy as jnp
from jax import lax
from jax.experimental import pallas as pl
from jax.experimental.pallas import tpu as pltpu
from jax.experimental.pallas import tpu_sc as plsc

B, L, C, T = 16, 512, 32, 4
D, R, NR, ND, K = 512, 128, 3, 3, 25
MB = 16
G = B // MB

_AT_B = (((0,), (0,)), ((), ()))


def _atb(a, b):
    return lax.dot_general(a, b, dimension_numbers=_AT_B,
                           preferred_element_type=jnp.float32)


def _stage1_body(x_ref, m_ref, xk_ref, Wv_ref, Wm_ref, be_ref,
                 regT_ref, reg_ref, Wp_ref, bp_ref,
                 Wft_ref, Wfb_ref, Ws_ref, Wt_ref, bf_ref, bs_ref, bt_ref,
                 Wc1_ref, bc1_ref, Wc2_ref, bc2_ref,
                 s_ref, df_ref, d2_ref, dp_ref, reg3_ref,
                 A_s, W2T_s, df_s, d2_s, dp_s):
    g = pl.program_id(0)

    @pl.when(g == 0)
    def _fold():
        Ws = Ws_ref[...]
        Wt = Wt_ref[...]
        A_s[...] = Wft_ref[...] @ jnp.concatenate([Ws, Wt - Ws], axis=1)
        Qm = Wfb_ref[...] @ Wt
        reg3 = reg_ref[...] @ Qm + (bf_ref[...] @ Wt + bs_ref[...] + bt_ref[...])
        reg3_ref[...] = jnp.concatenate(
            [reg3, jnp.zeros((R, 128 - C), jnp.float32)], axis=1)
        ri = lax.broadcasted_iota(jnp.int32, (L, L), 0)
        ci = lax.broadcasted_iota(jnp.int32, (L, L), 1)
        band = ((ci >= ri - 12) & (ci <= ri + 12)).astype(jnp.float32)
        ex0 = jnp.where(ri == 0, jnp.maximum(12 - ci, 0), 0).astype(jnp.float32)
        ex1 = jnp.where(ri == L - 1, jnp.maximum(ci - (L - 13), 0), 0).astype(jnp.float32)
        W2T_s[...] = (band + ex0 + ex1) * (1.0 / K)

    ones_col = jnp.full((L, 1), 1.0 / L, jnp.float32)
    for i in range(MB):
        xmT = x_ref[i] * m_ref[i]
        encT = jnp.tanh(_atb(Wv_ref[...], xmT) + _atb(Wm_ref[...], xk_ref[i])
                        + be_ref[...])
        df_col = jnp.mean(encT, axis=1, keepdims=True)
        df_row = lax.dot_general(ones_col, encT, (((0,), (1,)), ((), ())),
                                 preferred_element_type=jnp.float32)
        df_s[pl.ds(g * MB + i, 1)] = df_row
        uvT = _atb(A_s[...], encT)
        trendT = uvT[C:2 * C, :] @ W2T_s[...]
        s_ref[i] = uvT[0:C, :] + trendT
        xe_col = _atb(Wp_ref[...], df_col) + bp_ref[...]
        diffT = regT_ref[...] - xe_col
        d2_s[pl.ds(g * MB + i, 1)] = jnp.sum(diffT * diffT, axis=0)[None, :]
        h1 = jax.nn.relu(df_row @ Wc1_ref[...] + bc1_ref[...])
        dp_s[pl.ds(g * MB + i, 1)] = h1 @ Wc2_ref[...] + bc2_ref[...]

    @pl.when(g == G - 1)
    def _flush():
        df_ref[...] = df_s[...]
        d2_ref[...] = d2_s[...]
        dp_ref[...] = dp_s[...]


_VQ_SC_CACHE = []


def _get_vq_sc():
    if _VQ_SC_CACHE:
        return _VQ_SC_CACHE[0]
    mesh = plsc.VectorSubcoreMesh(core_axis_name="c", subcore_axis_name="s",
                                  num_cores=1)

    @functools.partial(
        pl.kernel,
        mesh=mesh,
        out_type=[jax.ShapeDtypeStruct((B, 128), jnp.float32),
                  jax.ShapeDtypeStruct((16,), jnp.float32)],
        scratch_types=[pltpu.VMEM((B, R), jnp.float32),
                       pltpu.VMEM((16, 128), jnp.float32),
                       pltpu.VMEM((16,), jnp.float32),
                       pltpu.SemaphoreType.DMA],
    )
    def _vq_sc(d2_hbm, reg3_hbm, cc_hbm, dmin_hbm,
               d2_v, rows_v, mv, sem):
        wid = lax.axis_index("c") * 16 + lax.axis_index("s")

        @pl.when(wid == 0)
        def _():
            pltpu.sync_copy(d2_hbm, d2_v)
            lane = lax.iota(jnp.int32, 16)
            dminv = jnp.zeros((16,), jnp.float32)
            idxv = jnp.zeros((16,), jnp.int32)
            for b in range(B):
                bestv = d2_v[b, pl.ds(0, 16)]
                besti = lane
                for c in range(1, R // 16):
                    v2 = d2_v[b, pl.ds(16 * c, 16)]
                    i2 = lane + 16 * c
                    upd = v2 < bestv
                    bestv = jnp.where(upd, v2, bestv)
                    besti = jnp.where(upd, i2, besti)
                for s in (1, 2, 4, 8):
                    perm = lane ^ s
                    v2 = bestv.at[perm].get(mode="promise_in_bounds")
                    i2 = besti.at[perm].get(mode="promise_in_bounds")
                    upd = (v2 < bestv) | ((v2 == bestv) & (i2 < besti))
                    bestv = jnp.where(upd, v2, bestv)
                    besti = jnp.where(upd, i2, besti)
                sel = lane == b
                dminv = jnp.where(sel, bestv, dminv)
                idxv = jnp.where(sel, besti, idxv)
            mv[...] = dminv
            pltpu.sync_copy(mv, dmin_hbm)
            pltpu.async_copy(reg3_hbm.at[idxv], rows_v, sem).wait()
            pltpu.sync_copy(rows_v, cc_hbm)

    _VQ_SC_CACHE.append(_vq_sc)
    return _vq_sc


def _stage3_body(s_ref, cc_ref, dmin_ref, x_ref, m_ref,
                 w1_ref, b1_ref, w2_ref, b2_ref,
                 out_ref, rf_ref, loss_ref, w1_s, w2_s):
    g = pl.program_id(0)

    @pl.when(g == 0)
    def _init():
        loss_ref[...] = jnp.sum(
            jnp.sqrt(dmin_ref[...]).reshape(1, 16), axis=1, keepdims=True) * (1.0 / B)
        for j in range(3):
            w1_s[j] = w1_ref[:, :, j]
            w2_s[j] = w2_ref[:, :, j]

    z1 = jnp.zeros((C, 1), jnp.float32)
    z2 = jnp.zeros((2 * C, 1), jnp.float32)
    for i in range(MB):
        ccT = jnp.transpose(cc_ref[i])
        reconT = s_ref[i] + ccT[0:C, :]
        rp = jnp.concatenate([z1, reconT, z1], axis=1)
        h = (w1_s[0] @ rp[:, 0:L] + w1_s[1] @ rp[:, 1:L + 1]
             + w1_s[2] @ rp[:, 2:L + 2] + b1_ref[...])
        h = jnp.maximum(h, 0.0)
        hp = jnp.concatenate([z2, h, z2], axis=1)
        r2 = (w2_s[0] @ hp[:, 0:L] + w2_s[1] @ hp[:, 1:L + 1]
              + w2_s[2] @ hp[:, 2:L + 2] + b2_ref[...])
        rf_ref[i] = r2
        out_ref[i] = m_ref[i] * x_ref[i] + (1.0 - m_ref[i]) * r2


def _const2(shape):
    return pl.BlockSpec(shape, lambda g: (0, 0))


def _make_stage1():
    f32 = jnp.float32
    return pl.pallas_call(
        _stage1_body,
        grid=(G,),
        in_specs=[
            pl.BlockSpec((MB, C, L), lambda g: (g, 0, 0)),
            pl.BlockSpec((MB, C, L), lambda g: (g, 0, 0)),
            pl.BlockSpec((MB, T, L), lambda g: (g, 0, 0)),
            _const2((C, D)),
            _const2((T, D)),
            _const2((D, 1)),
            _const2((D, R)),
            _const2((R, D)),
            _const2((D, D)),
            _const2((D, 1)),
            pl.BlockSpec((D, D), lambda g: (0, 0)),
            pl.BlockSpec((D, D), lambda g: (1, 0)),
            _const2((D, C)),
            _const2((D, C)),
            _const2((1, D)),
            _const2((1, C)),
            _const2((1, C)),
            _const2((D, D // 2)),
            _const2((1, D // 2)),
            _const2((D // 2, ND)),
            _const2((1, ND)),
        ],
        out_specs=[
            pl.BlockSpec((MB, C, L), lambda g: (g, 0, 0)),
            _const2((B, D)),
            _const2((B, R)),
            _const2((B, ND)),
            _const2((R, 128)),
        ],
        out_shape=[
            jax.ShapeDtypeStruct((B, C, L), f32),
            jax.ShapeDtypeStruct((B, D), f32),
            jax.ShapeDtypeStruct((B, R), f32),
            jax.ShapeDtypeStruct((B, ND), f32),
            jax.ShapeDtypeStruct((R, 128), f32),
        ],
        scratch_shapes=[pltpu.VMEM((D, 2 * C), f32),
                        pltpu.VMEM((L, L), f32),
                        pltpu.VMEM((B, D), f32),
                        pltpu.VMEM((B, R), f32),
                        pltpu.VMEM((B, ND), f32)],
    )


def _make_stage3():
    f32 = jnp.float32
    return pl.pallas_call(
        _stage3_body,
        grid=(G,),
        in_specs=[
            pl.BlockSpec((MB, C, L), lambda g: (g, 0, 0)),
            pl.BlockSpec((MB, 1, 128), lambda g: (g, 0, 0)),
            pl.BlockSpec((16,), lambda g: (0,)),
            pl.BlockSpec((MB, C, L), lambda g: (g, 0, 0)),
            pl.BlockSpec((MB, C, L), lambda g: (g, 0, 0)),
            pl.BlockSpec((2 * C, C, 3), lambda g: (0, 0, 0)),
            _const2((2 * C, 1)),
            pl.BlockSpec((C, 2 * C, 3), lambda g: (0, 0, 0)),
            _const2((C, 1)),
        ],
        out_specs=[
            pl.BlockSpec((MB, C, L), lambda g: (g, 0, 0)),
            pl.BlockSpec((MB, C, L), lambda g: (g, 0, 0)),
            _const2((1, 1)),
        ],
        out_shape=[
            jax.ShapeDtypeStruct((B, C, L), f32),
            jax.ShapeDtypeStruct((B, C, L), f32),
            jax.ShapeDtypeStruct((1, 1), f32),
        ],
        scratch_shapes=[pltpu.VMEM((3, 2 * C, C), f32),
                        pltpu.VMEM((3, C, 2 * C), f32)],
    )


def kernel(x_enc, x_mark_enc, mask, W_val, W_mark, b_enc, register, Wp, bp,
           Wf, bf, Ws, bs, Wt, bt, conv1_w, conv1_b, conv2_w, conv2_b,
           Wc1, bc1, Wc2, bc2):
    xT = jnp.transpose(x_enc, (0, 2, 1))
    mT = jnp.transpose(mask, (0, 2, 1))
    kT = jnp.transpose(x_mark_enc, (0, 2, 1))
    regT = jnp.transpose(register)
    be2 = b_enc.reshape(D, 1)
    bp2 = bp.reshape(D, 1)
    bf2 = bf.reshape(1, D)
    bs2 = bs.reshape(1, C)
    bt2 = bt.reshape(1, C)
    bc1_2 = bc1.reshape(1, D // 2)
    bc2_2 = bc2.reshape(1, ND)
    b1 = conv1_b.reshape(2 * C, 1)
    b2 = conv2_b.reshape(C, 1)

    sT, df, d2, dp, reg3 = _make_stage1()(
        xT, mT, kT, W_val, W_mark, be2, regT, register, Wp, bp2,
        Wf, Wf, Ws, Wt, bf2, bs2, bt2, Wc1, bc1_2, Wc2, bc2_2)

    cc, dmin = _get_vq_sc()(d2, reg3)

    outT, refT, loss11 = _make_stage3()(
        sT, cc.reshape(B, 1, 128), dmin, xT, mT, conv1_w, b1, conv2_w, b2)

    return (jnp.transpose(outT, (0, 2, 1)), jnp.transpose(refT, (0, 2, 1)),
            loss11.reshape(()), dp, df)
